# Initial kernel scaffold; baseline (speedup 1.0000x reference)
import jax
import jax.numpy as jnp
from jax.experimental import pallas as pl


def kernel(features, edge_index, edge_weight, W, bias):
    raise NotImplementedError("write your pallas kernel here")


# speedup vs baseline: 4.4693x; 4.4693x over previous
import functools

import jax
import jax.numpy as jnp
from jax import lax
from jax.experimental import pallas as pl
from jax.experimental.pallas import tpu as pltpu
from jax.experimental.pallas import tpu_sc as plsc

N = 10000
E = 320000
D = 128

NC = 2
NS = 16
L = 16
NW = NC * NS

EPW = E // NW
C = 80
NCHUNK = EPW // C
NP = 10240
RPS = NP // NS
ZROWS = 128

_mesh = plsc.VectorSubcoreMesh(core_axis_name="c", subcore_axis_name="s")


@functools.partial(
    pl.kernel,
    out_type=jax.ShapeDtypeStruct((NC, NP, D), jnp.float32),
    mesh=_mesh,
    scratch_types=[
        pltpu.VMEM((C,), jnp.int32),
        pltpu.VMEM((C,), jnp.int32),
        pltpu.VMEM((C,), jnp.float32),
        pltpu.VMEM((C, D), jnp.float32),
        pltpu.VMEM((ZROWS, D), jnp.float32),
        pltpu.VMEM_SHARED((NP, D), jnp.float32),
        pltpu.SemaphoreType.DMA,
    ],
)
def _sc_spmm(support_hbm, src_hbm, dst_hbm, w_hbm, out_hbm,
             src_v, dst_v, w_v, rows_v, zeros_v, acc, sem):
    cid = lax.axis_index("c")
    sid = lax.axis_index("s")
    wid = sid * NC + cid

    def _zrow(i, carry):
        for g in range(D // L):
            zeros_v[i, pl.ds(g * L, L)] = jnp.zeros((L,), jnp.float32)
        return carry
    lax.fori_loop(0, ZROWS, _zrow, 0)

    base_row = sid * RPS
    for r in range(RPS // ZROWS):
        pltpu.sync_copy(zeros_v, acc.at[pl.ds(base_row + r * ZROWS, ZROWS)])

    plsc.subcore_barrier()

    ebase = wid * EPW

    def _chunk(j, carry):
        off = pl.multiple_of(ebase + j * C, 8)
        pltpu.sync_copy(src_hbm.at[pl.ds(off, C)], src_v)
        pltpu.sync_copy(dst_hbm.at[pl.ds(off, C)], dst_v)
        pltpu.sync_copy(w_hbm.at[pl.ds(off, C)], w_v)
        pltpu.async_copy(support_hbm.at[src_v], rows_v, sem).wait()

        def _egroup(eg, c2):
            wvec = w_v[pl.ds(eg * L, L)]
            for k in range(L):
                wv = wvec[k]
                e = eg * L + k
                for g in range(D // L):
                    rows_v[e, pl.ds(g * L, L)] = rows_v[e, pl.ds(g * L, L)] * wv
            return c2
        lax.fori_loop(0, C // L, _egroup, 0)

        pltpu.sync_copy(rows_v, acc.at[dst_v], add=True)
        return carry

    lax.fori_loop(0, NCHUNK, _chunk, 0)

    plsc.subcore_barrier()

    pltpu.sync_copy(acc.at[pl.ds(base_row, RPS)],
                    out_hbm.at[cid, pl.ds(base_row, RPS)])


def _mm_body(x_ref, w_ref, o_ref):
    o_ref[...] = jnp.dot(x_ref[...], w_ref[...],
                         preferred_element_type=jnp.float32)


def _comb_body(p_ref, b_ref, o_ref):
    o_ref[...] = p_ref[0] + p_ref[1] + b_ref[...]


_MM_BLK = 1000


def _tc_matmul(features, W):
    return pl.pallas_call(
        _mm_body,
        grid=(N // _MM_BLK,),
        in_specs=[
            pl.BlockSpec((_MM_BLK, D), lambda i: (i, 0)),
            pl.BlockSpec((D, D), lambda i: (0, 0)),
        ],
        out_specs=pl.BlockSpec((_MM_BLK, D), lambda i: (i, 0)),
        out_shape=jax.ShapeDtypeStruct((N, D), jnp.float32),
    )(features, W)


def _tc_combine(partials, bias):
    return pl.pallas_call(
        _comb_body,
        grid=(N // _MM_BLK,),
        in_specs=[
            pl.BlockSpec((NC, _MM_BLK, D), lambda i: (0, i, 0)),
            pl.BlockSpec((1, D), lambda i: (0, 0)),
        ],
        out_specs=pl.BlockSpec((_MM_BLK, D), lambda i: (i, 0)),
        out_shape=jax.ShapeDtypeStruct((N, D), jnp.float32),
    )(partials, bias[None, :])


@jax.jit
def kernel(features, edge_index, edge_weight, W, bias):
    src = edge_index[0]
    dst = edge_index[1]
    support = _tc_matmul(features, W)
    partials = _sc_spmm(support, src, dst, edge_weight)
    return _tc_combine(partials, bias)


# speedup vs baseline: 9.2948x; 2.0797x over previous
import functools

import jax
import jax.numpy as jnp
from jax import lax
from jax.experimental import pallas as pl
from jax.experimental.pallas import tpu as pltpu
from jax.experimental.pallas import tpu_sc as plsc

N = 10000
E = 320000
D = 128

NC = 2
NS = 16
L = 16
NW = NC * NS

EPW = E // NW
C = 80
SS = 5
SC_CH = 25
SC_E = SC_CH * C
RPS = 624

_mesh = plsc.VectorSubcoreMesh(core_axis_name="c", subcore_axis_name="s")


@functools.partial(
    pl.kernel,
    out_type=jax.ShapeDtypeStruct((NC, N, D), jnp.float32),
    mesh=_mesh,
    scratch_types=[
        pltpu.VMEM((SC_E,), jnp.int32),
        pltpu.VMEM((SC_E,), jnp.int32),
        pltpu.VMEM((SC_CH, C), jnp.int32),
        pltpu.VMEM((SC_CH, C), jnp.int32),
        pltpu.VMEM((SC_E,), jnp.float32),
        pltpu.VMEM((SC_E,), jnp.float32),
        pltpu.VMEM((C, D), jnp.float32),
        pltpu.VMEM((C, D), jnp.float32),
        pltpu.VMEM((C, D), jnp.float32),
        pltpu.VMEM_SHARED((N, D), jnp.float32),
        pltpu.SemaphoreType.DMA,
        pltpu.SemaphoreType.DMA,
        pltpu.SemaphoreType.DMA,
        pltpu.SemaphoreType.DMA,
        pltpu.SemaphoreType.DMA,
        pltpu.SemaphoreType.DMA,
        pltpu.SemaphoreType.DMA,
    ],
)
def _sc_spmm(support_hbm, src_hbm, dst_hbm, w_hbm, out_hbm,
             src_a, src_b, dst_a, dst_b, w_a, w_b,
             buf0, buf1, buf2, acc,
             gs0, gs1, gs2, ss0, ss1, ss2, sem_i):
    cid = lax.axis_index("c")
    sid = lax.axis_index("s")
    wid = sid * NC + cid
    lastsub = sid == NS - 1

    bufs = (buf0, buf1, buf2)
    gsems = (gs0, gs1, gs2)
    ssems = (ss0, ss1, ss2)

    def _zrow(i, carry):
        for g in range(D // L):
            buf0[i, pl.ds(g * L, L)] = jnp.zeros((L,), jnp.float32)
            buf1[i, pl.ds(g * L, L)] = jnp.zeros((L,), jnp.float32)
            buf2[i, pl.ds(g * L, L)] = jnp.zeros((L,), jnp.float32)
        return carry
    lax.fori_loop(0, C, _zrow, 0)

    pltpu.sync_copy(src_hbm.at[wid, 0, 0], src_a)
    pltpu.sync_copy(dst_hbm.at[wid, 0], dst_a)
    pltpu.sync_copy(w_hbm.at[wid, 0, 0], w_a)

    base_row = sid * RPS
    pltpu.async_copy(buf1, acc.at[pl.ds(base_row, C)], ss1)
    pltpu.async_copy(buf2, acc.at[pl.ds(base_row, C)], ss2)

    def _zacc(z, carry):
        pltpu.sync_copy(buf0, acc.at[pl.ds(base_row + z * C, C)])
        return carry
    lax.fori_loop(0, 7, _zacc, 0)

    @pl.when(lastsub)
    def _():
        pltpu.sync_copy(buf0, acc.at[pl.ds(base_row + 560, C)])

    @pl.when(jnp.logical_not(lastsub))
    def _():
        pltpu.sync_copy(buf0.at[pl.ds(0, 64)],
                        acc.at[pl.ds(base_row + 560, 64)])

    def _gstart(buf, src_s, lc, sem):
        off = pl.multiple_of(lc * C, 8)
        pltpu.async_copy(support_hbm.at[src_s.at[pl.ds(off, C)]], buf, sem)

    def _gwait(buf, sem):
        pltpu.make_async_copy(support_hbm.at[src_a.at[pl.ds(0, C)]],
                              buf, sem).wait()

    def _scale(buf, w_s, lc):
        def _egroup(eg, c2):
            wvec = w_s[pl.ds(lc * C + eg * L, L)]
            for k in range(L):
                wv = wvec[k]
                e = eg * L + k
                for g in range(D // L):
                    buf[e, pl.ds(g * L, L)] = buf[e, pl.ds(g * L, L)] * wv
            return c2
        lax.fori_loop(0, C // L, _egroup, 0)

    def _fire(buf, dst_s, lc, sem):
        pltpu.async_copy(buf, acc.at[dst_s.at[lc]], sem, add=True)

    def _drain(buf, sem):
        pltpu.make_async_copy(buf, acc.at[pl.ds(0, C)], sem).wait()

    def _idx_drain(src_n):
        d = pltpu.make_async_copy(src_hbm.at[wid, 0, 0], src_n, sem_i)
        d.wait()
        d.wait()
        d.wait()

    _gstart(buf0, src_a, 0, gs0)

    plsc.subcore_barrier()

    slots = ((src_a, dst_a, w_a), (src_b, dst_b, w_b))

    for s in range(SS):
        src_s, dst_s, w_s = slots[s % 2]
        src_n, dst_n, w_n = slots[(s + 1) % 2]

        def _step(lc, bi, pre=None, src_s=src_s, dst_s=dst_s, w_s=w_s):
            B = bufs[bi]
            Bn = bufs[(bi + 1) % 3]
            _gwait(B, gsems[bi])
            _drain(Bn, ssems[(bi + 1) % 3])
            if pre is not None:
                pre()
            _gstart(Bn, src_s, lc + 1, gsems[(bi + 1) % 3])
            _scale(B, w_s, lc)
            _fire(B, dst_s, lc, ssems[bi])

        def _pref(src_n=src_n, dst_n=dst_n, w_n=w_n, s=s):
            pltpu.async_copy(src_hbm.at[wid, s + 1, 0], src_n, sem_i)
            pltpu.async_copy(dst_hbm.at[wid, s + 1], dst_n, sem_i)
            pltpu.async_copy(w_hbm.at[wid, s + 1, 0], w_n, sem_i)

        def _triple(t, carry, s=s):
            lc = t * 3
            _step(lc, s % 3)
            if s < SS - 1:
                def pre2():
                    @pl.when(t == 0)
                    def _():
                        _pref()
                _step(lc + 1, (s + 1) % 3, pre=pre2)
            else:
                _step(lc + 1, (s + 1) % 3)
            _step(lc + 2, (s + 2) % 3)
            return carry

        lax.fori_loop(0, SC_CH // 3, _triple, 0)

        bi = s % 3
        B = bufs[bi]
        _gwait(B, gsems[bi])
        _drain(bufs[(bi + 1) % 3], ssems[(bi + 1) % 3])
        if s < SS - 1:
            _idx_drain(src_n)
            _gstart(bufs[(bi + 1) % 3], src_n, 0, gsems[(bi + 1) % 3])
        _scale(B, w_s, SC_CH - 1)
        _fire(B, dst_s, SC_CH - 1, ssems[bi])

    _drain(bufs[0], ssems[0])
    _drain(bufs[1], ssems[1])

    plsc.subcore_barrier()

    pltpu.sync_copy(acc.at[pl.ds(base_row, RPS)],
                    out_hbm.at[cid, pl.ds(base_row, RPS)])

    @pl.when(lastsub)
    def _():
        pltpu.sync_copy(acc.at[pl.ds(N - L, L)],
                        out_hbm.at[cid, pl.ds(N - L, L)])


def _mm_body(x_ref, w_ref, o_ref):
    o_ref[...] = jnp.dot(x_ref[...], w_ref[...],
                         preferred_element_type=jnp.float32)


def _comb_body(p_ref, b_ref, o_ref):
    o_ref[...] = p_ref[0] + p_ref[1] + b_ref[...]


_MM_BLK = 1000


def _tc_matmul(features, W):
    return pl.pallas_call(
        _mm_body,
        grid=(N // _MM_BLK,),
        in_specs=[
            pl.BlockSpec((_MM_BLK, D), lambda i: (i, 0)),
            pl.BlockSpec((D, D), lambda i: (0, 0)),
        ],
        out_specs=pl.BlockSpec((_MM_BLK, D), lambda i: (i, 0)),
        out_shape=jax.ShapeDtypeStruct((N, D), jnp.float32),
    )(features, W)


def _tc_combine(partials, bias):
    return pl.pallas_call(
        _comb_body,
        grid=(N // _MM_BLK,),
        in_specs=[
            pl.BlockSpec((NC, _MM_BLK, D), lambda i: (0, i, 0)),
            pl.BlockSpec((1, D), lambda i: (0, 0)),
        ],
        out_specs=pl.BlockSpec((_MM_BLK, D), lambda i: (i, 0)),
        out_shape=jax.ShapeDtypeStruct((N, D), jnp.float32),
    )(partials, bias[None, :])


@jax.jit
def kernel(features, edge_index, edge_weight, W, bias):
    src = edge_index[0].reshape(NW, SS, 1, SC_E)
    dst = edge_index[1].reshape(NW, SS, SC_CH, C)
    w3 = edge_weight.reshape(NW, SS, 1, SC_E)
    support = _tc_matmul(features, W)
    partials = _sc_spmm(support, src, dst, w3)
    return _tc_combine(partials, bias)


# speedup vs baseline: 9.3489x; 1.0058x over previous
import functools

import jax
import jax.numpy as jnp
from jax import lax
from jax.experimental import pallas as pl
from jax.experimental.pallas import tpu as pltpu
from jax.experimental.pallas import tpu_sc as plsc

N = 10000
E = 320000
D = 128

NC = 2
NS = 16
L = 16
NW = NC * NS

EPW = E // NW
C = 80
SS = 5
SC_CH = 25
SC_E = SC_CH * C
RPS = 624

_mesh = plsc.VectorSubcoreMesh(core_axis_name="c", subcore_axis_name="s")


@functools.partial(
    pl.kernel,
    out_type=jax.ShapeDtypeStruct((NC, N, D), jnp.float32),
    mesh=_mesh,
    scratch_types=[
        pltpu.VMEM((SC_E,), jnp.int32),
        pltpu.VMEM((SC_E,), jnp.int32),
        pltpu.VMEM((SC_CH, C), jnp.int32),
        pltpu.VMEM((SC_CH, C), jnp.int32),
        pltpu.VMEM((SC_E,), jnp.float32),
        pltpu.VMEM((SC_E,), jnp.float32),
        pltpu.VMEM((C, D), jnp.float32),
        pltpu.VMEM((C, D), jnp.float32),
        pltpu.VMEM_SHARED((N, D), jnp.float32),
        pltpu.SemaphoreType.DMA,
        pltpu.SemaphoreType.DMA,
        pltpu.SemaphoreType.DMA,
        pltpu.SemaphoreType.DMA,
        pltpu.SemaphoreType.DMA,
    ],
)
def _sc_spmm(support_hbm, src_hbm, dst_hbm, w_hbm, out_hbm,
             src_a, src_b, dst_a, dst_b, w_a, w_b, buf_a, buf_b, acc,
             sem_ga, sem_gb, sem_sa, sem_sb, sem_i):
    cid = lax.axis_index("c")
    sid = lax.axis_index("s")
    wid = sid * NC + cid
    lastsub = sid == NS - 1

    def _zrow(i, carry):
        for g in range(D // L):
            buf_a[i, pl.ds(g * L, L)] = jnp.zeros((L,), jnp.float32)
            buf_b[i, pl.ds(g * L, L)] = jnp.zeros((L,), jnp.float32)
        return carry
    lax.fori_loop(0, C, _zrow, 0)

    pltpu.sync_copy(src_hbm.at[wid, 0, 0], src_a)
    pltpu.sync_copy(dst_hbm.at[wid, 0], dst_a)
    pltpu.sync_copy(w_hbm.at[wid, 0, 0], w_a)

    base_row = sid * RPS
    pltpu.async_copy(buf_b, acc.at[pl.ds(base_row, C)], sem_sb)

    def _zacc(z, carry):
        pltpu.sync_copy(buf_a, acc.at[pl.ds(base_row + z * C, C)])
        return carry
    lax.fori_loop(0, 7, _zacc, 0)

    @pl.when(lastsub)
    def _():
        pltpu.sync_copy(buf_a, acc.at[pl.ds(base_row + 560, C)])

    @pl.when(jnp.logical_not(lastsub))
    def _():
        pltpu.sync_copy(buf_a.at[pl.ds(0, 64)],
                        acc.at[pl.ds(base_row + 560, 64)])

    plsc.subcore_barrier()

    def _gstart(buf, src_s, lc, sem):
        off = pl.multiple_of(lc * C, 8)
        pltpu.async_copy(support_hbm.at[src_s.at[pl.ds(off, C)]], buf, sem)

    def _gwait(buf, sem):
        pltpu.make_async_copy(support_hbm.at[src_a.at[pl.ds(0, C)]],
                              buf, sem).wait()

    def _scale(buf, w_s, lc):
        def _egroup(eg, c2):
            wvec = w_s[pl.ds(lc * C + eg * L, L)]
            for k in range(L):
                wv = wvec[k]
                e = eg * L + k
                for g in range(D // L):
                    buf[e, pl.ds(g * L, L)] = buf[e, pl.ds(g * L, L)] * wv
            return c2
        lax.fori_loop(0, C // L, _egroup, 0)

    def _fire(buf, dst_s, lc, sem):
        pltpu.async_copy(buf, acc.at[dst_s.at[lc]], sem, add=True)

    def _drain(buf, sem):
        pltpu.make_async_copy(buf, acc.at[pl.ds(0, C)], sem).wait()

    def _idx_drain(src_n):
        d = pltpu.make_async_copy(src_hbm.at[wid, 0, 0], src_n, sem_i)
        d.wait()
        d.wait()
        d.wait()

    _gstart(buf_a, src_a, 0, sem_ga)

    bufs = (buf_a, buf_b)
    gsems = (sem_ga, sem_gb)
    ssems = (sem_sa, sem_sb)
    slots = ((src_a, dst_a, w_a), (src_b, dst_b, w_b))

    for s in range(SS):
        p = s % 2
        q = 1 - p
        P, Q = bufs[p], bufs[q]
        gP, gQ = gsems[p], gsems[q]
        sP, sQ = ssems[p], ssems[q]
        src_s, dst_s, w_s = slots[p]
        src_n, dst_n, w_n = slots[q]

        def _pair(t, carry, P=P, Q=Q, gP=gP, gQ=gQ, sP=sP, sQ=sQ,
                  src_s=src_s, dst_s=dst_s, w_s=w_s,
                  src_n=src_n, dst_n=dst_n, w_n=w_n, s=s):
            lc = t * 2
            _gwait(P, gP)
            _drain(Q, sQ)

            if s < SS - 1:
                @pl.when(t == 0)
                def _():
                    pltpu.async_copy(src_hbm.at[wid, s + 1, 0], src_n, sem_i)
                    pltpu.async_copy(dst_hbm.at[wid, s + 1], dst_n, sem_i)
                    pltpu.async_copy(w_hbm.at[wid, s + 1, 0], w_n, sem_i)

            _gstart(Q, src_s, lc + 1, gQ)
            _scale(P, w_s, lc)
            _fire(P, dst_s, lc, sP)
            _gwait(Q, gQ)
            _drain(P, sP)
            _gstart(P, src_s, lc + 2, gP)
            _scale(Q, w_s, lc + 1)
            _fire(Q, dst_s, lc + 1, sQ)
            return carry

        lax.fori_loop(0, SC_CH // 2, _pair, 0)

        _gwait(P, gP)
        _drain(Q, sQ)
        if s < SS - 1:
            _idx_drain(src_n)
            _gstart(Q, src_n, 0, gQ)
        _scale(P, w_s, SC_CH - 1)
        _fire(P, dst_s, SC_CH - 1, sP)

    _drain(bufs[(SS - 1) % 2], ssems[(SS - 1) % 2])

    plsc.subcore_barrier()

    pltpu.sync_copy(acc.at[pl.ds(base_row, RPS)],
                    out_hbm.at[cid, pl.ds(base_row, RPS)])

    @pl.when(lastsub)
    def _():
        pltpu.sync_copy(acc.at[pl.ds(N - L, L)],
                        out_hbm.at[cid, pl.ds(N - L, L)])


def _mm_body(x_ref, w_ref, o_ref):
    o_ref[...] = jnp.dot(x_ref[...], w_ref[...],
                         preferred_element_type=jnp.float32)


def _comb_body(p_ref, b_ref, o_ref):
    o_ref[...] = p_ref[0] + p_ref[1] + b_ref[...]


_MM_BLK = 1000


def _tc_matmul(features, W):
    return pl.pallas_call(
        _mm_body,
        grid=(N // _MM_BLK,),
        in_specs=[
            pl.BlockSpec((_MM_BLK, D), lambda i: (i, 0)),
            pl.BlockSpec((D, D), lambda i: (0, 0)),
        ],
        out_specs=pl.BlockSpec((_MM_BLK, D), lambda i: (i, 0)),
        out_shape=jax.ShapeDtypeStruct((N, D), jnp.float32),
    )(features, W)


def _tc_combine(partials, bias):
    return pl.pallas_call(
        _comb_body,
        grid=(N // _MM_BLK,),
        in_specs=[
            pl.BlockSpec((NC, _MM_BLK, D), lambda i: (0, i, 0)),
            pl.BlockSpec((1, D), lambda i: (0, 0)),
        ],
        out_specs=pl.BlockSpec((_MM_BLK, D), lambda i: (i, 0)),
        out_shape=jax.ShapeDtypeStruct((N, D), jnp.float32),
    )(partials, bias[None, :])


@jax.jit
def kernel(features, edge_index, edge_weight, W, bias):
    src = edge_index[0].reshape(NW, SS, 1, SC_E)
    dst = edge_index[1].reshape(NW, SS, SC_CH, C)
    w3 = edge_weight.reshape(NW, SS, 1, SC_E)
    support = _tc_matmul(features, W)
    partials = _sc_spmm(support, src, dst, w3)
    return _tc_combine(partials, bias)
